# baseline (speedup 1.0000x reference)
import jax
import jax.numpy as jnp
from jax.experimental import pallas as pl
from jax.experimental.pallas import tpu as pltpu

B, T, K, D = 16, 2048, 1024, 256
UPAD = 2056


def _loss_kernel(ul_ref, units_ref, c_hi_ref, c_lo_ref, cent_ref, out_ref,
                 u_ref, p_ref):
    b = pl.program_id(0)
    L = ul_ref[b]

    nchunks = (L + 255) >> 8

    def mm_body(j, carry):
        u2 = units_ref[0, pl.ds(j * 256, 256), :]
        oh = (u2 == jax.lax.broadcasted_iota(jnp.int32, (256, D), 1))
        ohb = oh.astype(jnp.bfloat16)
        u_ref[pl.ds(j * 256, 256), :] = (
            jnp.dot(ohb, c_hi_ref[...], preferred_element_type=jnp.float32)
            + jnp.dot(ohb, c_lo_ref[...], preferred_element_type=jnp.float32)
        )
        return carry

    jax.lax.fori_loop(0, nchunks, mm_body, 0)

    p_ref[:, 0, :] = cent_ref[0, 0:1024, :].T

    @pl.when(L > 1024)
    def _():
        p_ref[:, 1, :] = cent_ref[0, 1024:2048, :].T

    lane2 = jax.lax.broadcasted_iota(jnp.int32, (2, 1024), 1)
    t3 = (jax.lax.broadcasted_iota(jnp.int32, (3, 1024), 0) * 1024
          + jax.lax.broadcasted_iota(jnp.int32, (3, 1024), 1))

    def body(k, acc):
        base = k * L
        sr = base >> 10
        s = base & 1023
        w = jnp.concatenate(
            [u_ref[pl.ds(sr + i, 1), :] for i in range(3)], axis=0)
        prow = p_ref[k]
        r = pltpu.roll(prow, s, axis=1)
        rswap = jnp.concatenate([r[1:2], r[0:1]], axis=0)
        prot = jnp.where(lane2 >= s, r, rswap)
        pp = jnp.concatenate([prot, prot[0:1]], axis=0)
        mask = (t3 >= s) & (t3 < (s + L))
        return acc + jnp.where(mask, jnp.abs(w - pp), 0.0)

    acc = jax.lax.fori_loop(
        0, K, body, jnp.zeros((3, 1024), jnp.float32), unroll=4)
    total = jnp.sum(acc)

    @pl.when(b == 0)
    def _():
        out_ref[0, 0] = 0.0

    out_ref[0, 0] += total / (L.astype(jnp.float32) * B)


@jax.jit
def kernel(centroids, units, unit_lengths, C):
    c_hi = C.astype(jnp.bfloat16)
    c_lo = (C - c_hi.astype(jnp.float32)).astype(jnp.bfloat16)
    units3 = units.reshape(B, T, 1)

    out = pl.pallas_call(
        _loss_kernel,
        grid=(B,),
        in_specs=[
            pl.BlockSpec(memory_space=pltpu.SMEM),
            pl.BlockSpec((1, T, 1), lambda b: (b, 0, 0)),
            pl.BlockSpec((D, K), lambda b: (0, 0)),
            pl.BlockSpec((D, K), lambda b: (0, 0)),
            pl.BlockSpec((1, T, K), lambda b: (b, 0, 0)),
        ],
        out_specs=pl.BlockSpec(memory_space=pltpu.SMEM),
        out_shape=jax.ShapeDtypeStruct((1, 1), jnp.float32),
        scratch_shapes=[
            pltpu.VMEM((UPAD, K), jnp.float32),
            pltpu.VMEM((K, 2, 1024), jnp.float32),
        ],
    )(unit_lengths, units3, c_hi, c_lo, centroids)
    return out[0, 0]


# speedup vs baseline: 1.4774x; 1.4774x over previous
import jax
import jax.numpy as jnp
from jax.experimental import pallas as pl
from jax.experimental.pallas import tpu as pltpu

B, T, K, D = 16, 2048, 1024, 256
UPAD = 2056


def _loss_kernel(ul_ref, units_ref, c_hi_ref, c_lo_ref, cent_ref, out_ref,
                 u_ref, p_ref):
    b = pl.program_id(0)
    L = ul_ref[b]

    nchunks = (L + 255) >> 8

    def mm_body(j, carry):
        u2 = units_ref[0, pl.ds(j * 256, 256), :]
        oh = (u2 == jax.lax.broadcasted_iota(jnp.int32, (256, D), 1))
        ohb = oh.astype(jnp.bfloat16)
        u_ref[pl.ds(j * 256, 256), :] = (
            jnp.dot(ohb, c_hi_ref[...], preferred_element_type=jnp.float32)
            + jnp.dot(ohb, c_lo_ref[...], preferred_element_type=jnp.float32)
        )
        return carry

    jax.lax.fori_loop(0, nchunks, mm_body, 0)

    p_ref[:, 0, :] = cent_ref[0, 0:1024, :].T

    @pl.when(L > 1024)
    def _():
        p_ref[:, 1, :] = cent_ref[0, 1024:2048, :].T

    lane2 = jax.lax.broadcasted_iota(jnp.int32, (2, 1024), 1)
    tmat = (jax.lax.broadcasted_iota(jnp.int32, (2, 1024), 0) * 1024 + lane2)
    tmask = tmat < L

    def body(k, acc):
        base = k * L
        sr = base >> 10
        s = base & 1023
        w = jnp.concatenate(
            [u_ref[pl.ds(sr + i, 1), :] for i in range(3)], axis=0)
        rolled = pltpu.roll(w, 1024 - s, axis=1)
        sel = lane2 < (1024 - s)
        out = jnp.where(sel, rolled[0:2, :], rolled[1:3, :])
        diff = jnp.where(tmask, jnp.abs(p_ref[k] - out), 0.0)
        return acc + diff

    acc = jax.lax.fori_loop(
        0, K, body, jnp.zeros((2, 1024), jnp.float32), unroll=8)
    total = jnp.sum(acc)

    @pl.when(b == 0)
    def _():
        out_ref[0, 0] = 0.0

    out_ref[0, 0] += total / (L.astype(jnp.float32) * B)


@jax.jit
def kernel(centroids, units, unit_lengths, C):
    c_hi = C.astype(jnp.bfloat16)
    c_lo = (C - c_hi.astype(jnp.float32)).astype(jnp.bfloat16)
    units3 = units.reshape(B, T, 1)

    out = pl.pallas_call(
        _loss_kernel,
        grid=(B,),
        in_specs=[
            pl.BlockSpec(memory_space=pltpu.SMEM),
            pl.BlockSpec((1, T, 1), lambda b: (b, 0, 0)),
            pl.BlockSpec((D, K), lambda b: (0, 0)),
            pl.BlockSpec((D, K), lambda b: (0, 0)),
            pl.BlockSpec((1, T, K), lambda b: (b, 0, 0)),
        ],
        out_specs=pl.BlockSpec(memory_space=pltpu.SMEM),
        out_shape=jax.ShapeDtypeStruct((1, 1), jnp.float32),
        scratch_shapes=[
            pltpu.VMEM((UPAD, K), jnp.float32),
            pltpu.VMEM((K, 2, 1024), jnp.float32),
        ],
    )(unit_lengths, units3, c_hi, c_lo, centroids)
    return out[0, 0]
